# baseline (speedup 1.0000x reference)
import functools

import jax
import jax.numpy as jnp
import numpy as np
from jax import lax
from jax.experimental import pallas as pl
from jax.experimental.pallas import tpu as pltpu
from jax.experimental.pallas import tpu_sc as plsc

L = 16
T = 131072
F = 8
D = 256
N = 16384

_rng = np.random.RandomState(1234)
_PRIMES = _rng.randint(1, 2**31 - 1, size=(D,)).astype(np.uint32).astype(np.int32)
_SCALES = (2.0 ** np.linspace(4.0, 12.0, L)).astype(np.float32)

_MASK = T - 1

_EXPAND = (np.arange(128)[None, :] // F == np.arange(L)[:, None]).astype(np.float32)

_NC = 2
_NS = 16
_NW = _NC * _NS
_ROWS = N * L
_RPW = _ROWS // _NW
_CHUNK = 128
_NCHUNK = _RPW // _CHUNK


def _hash_body(z_ref, primes_ref, e_ref, out_ref):
    zt = z_ref[...].T
    primes = primes_ref[...]
    rows = []
    for l in range(L):
        q = (zt * _SCALES[l]).astype(jnp.int32)
        h = q * primes
        s = jnp.sum(h, axis=0) & _MASK
        rows.append((l * (T * F)) + ((s >> 7) << 10) + (s & 127))
    bt = jnp.stack(rows, axis=0).T.astype(jnp.float32)
    exp = jnp.dot(bt, e_ref[...], preferred_element_type=jnp.float32,
                  precision=lax.Precision.HIGHEST)
    ftile = (jnp.arange(128, dtype=jnp.int32)[None, :] & 7) * 128
    out_ref[...] = exp.astype(jnp.int32) + ftile


def _mlp_body(enc_ref, w1_ref, b1_ref, w2_ref, b2_ref, out_ref):
    e = enc_ref[...]
    h = jnp.dot(e, w1_ref[...], preferred_element_type=jnp.float32) + b1_ref[...]
    h = jnp.where(h > 0, h, 0.01 * h)
    out_ref[...] = jnp.dot(h, w2_ref[...],
                           preferred_element_type=jnp.float32) + b2_ref[...]


_NSLICE = 4
_NS = N // _NSLICE
_TPW = _NS // _NW
_HALF = _TPW // 2


@functools.lru_cache(maxsize=None)
def _make_gather_sc():
    @functools.partial(
        pl.kernel,
        mesh=plsc.VectorSubcoreMesh(core_axis_name="c", subcore_axis_name="s"),
        compiler_params=pltpu.CompilerParams(use_tc_tiling_on_sc=False),
        out_type=jax.ShapeDtypeStruct((_NW, _TPW, 128), jnp.float32),
        scratch_types=[
            pltpu.VMEM((_HALF, 128), jnp.int32),
            pltpu.VMEM((_TPW, 128), jnp.float32),
            pltpu.SemaphoreType.DMA,
        ],
    )
    def _gather_sc(tab_hbm, idx_hbm, out_hbm, idx_v, rows_v, sem):
        wid = lax.axis_index("s") * _NC + lax.axis_index("c")
        K = 32

        for p in range(2):
            pltpu.sync_copy(idx_hbm.at[wid, pl.ds(p * _HALF, _HALF)], idx_v)
            base = p * _HALF

            def fire(j):
                pltpu.make_async_copy(
                    tab_hbm.at[idx_v.at[j]], rows_v.at[base + j], sem
                ).start()

            def drain(j):
                pltpu.make_async_copy(
                    tab_hbm.at[idx_v.at[j]], rows_v.at[base + j], sem
                ).wait()

            for k in range(K):
                fire(k)

            def body(jj, carry):
                j = jj * 2
                drain(j)
                fire(j + K)
                drain(j + 1)
                fire(j + 1 + K)
                return carry

            lax.fori_loop(0, (_HALF - K) // 2, body, 0)
            for k in range(_HALF - K, _HALF):
                drain(k)
        pltpu.sync_copy(rows_v, out_hbm.at[wid])

    return _gather_sc


def kernel(z, tables, W1, b1, W2, b2):
    HB = 512
    primes = jnp.asarray(_PRIMES).reshape(D, 1)
    expand = jnp.asarray(_EXPAND)

    tab_lin = tables.reshape(L, T // 128, 128, F).transpose(0, 1, 3, 2)
    tab_flat = tab_lin.reshape(L * T * F)

    gather = _make_gather_sc()
    enc_slices = []
    for s in range(_NSLICE):
        idx_s = pl.pallas_call(
            _hash_body,
            grid=(_NS // HB,),
            in_specs=[
                pl.BlockSpec((HB, D), lambda i, s=s: (s * (_NS // HB) + i, 0)),
                pl.BlockSpec((D, 1), lambda i: (0, 0)),
                pl.BlockSpec((L, 128), lambda i: (0, 0)),
            ],
            out_specs=pl.BlockSpec((HB, 128), lambda i: (i, 0)),
            out_shape=jax.ShapeDtypeStruct((_NS, 128), jnp.int32),
        )(z, primes, expand)
        rows_s = gather(tab_flat, idx_s.reshape(_NW, _TPW, 128))
        enc_slices.append(rows_s.reshape(_NS, L * F))
    enc = jnp.concatenate(enc_slices, axis=0)

    MB = 1024
    out = pl.pallas_call(
        _mlp_body,
        grid=(N // MB,),
        in_specs=[
            pl.BlockSpec((MB, 128), lambda i: (i, 0)),
            pl.BlockSpec((128, 128), lambda i: (0, 0)),
            pl.BlockSpec((1, 128), lambda i: (0, 0)),
            pl.BlockSpec((128, 256), lambda i: (0, 0)),
            pl.BlockSpec((1, 256), lambda i: (0, 0)),
        ],
        out_specs=pl.BlockSpec((MB, 256), lambda i: (i, 0)),
        out_shape=jax.ShapeDtypeStruct((N, 256), jnp.float32),
    )(enc, W1, b1.reshape(1, 128), W2, b2.reshape(1, 256))
    return out


# speedup vs baseline: 1.1040x; 1.1040x over previous
import functools

import jax
import jax.numpy as jnp
import numpy as np
from jax import lax
from jax.experimental import pallas as pl
from jax.experimental.pallas import tpu as pltpu
from jax.experimental.pallas import tpu_sc as plsc

L = 16
T = 131072
F = 8
D = 256
N = 16384

_rng = np.random.RandomState(1234)
_PRIMES = _rng.randint(1, 2**31 - 1, size=(D,)).astype(np.uint32).astype(np.int32)
_SCALES = (2.0 ** np.linspace(4.0, 12.0, L)).astype(np.float32)

_MASK = T - 1

_EXPAND = (np.arange(128)[None, :] // F == np.arange(L)[:, None]).astype(np.float32)

_NC = 2
_NS = 16
_NW = _NC * _NS
_ROWS = N * L
_RPW = _ROWS // _NW
_CHUNK = 128
_NCHUNK = _RPW // _CHUNK


def _hash_body(z_ref, primes_ref, e_ref, out_ref):
    zt = z_ref[...].T
    primes = primes_ref[...]
    rows = []
    for l in range(L):
        q = (zt * _SCALES[l]).astype(jnp.int32)
        h = q * primes
        s = jnp.sum(h, axis=0) & _MASK
        rows.append((l * (T * F)) + ((s >> 7) << 10) + (s & 127))
    bt = jnp.stack(rows, axis=0).T.astype(jnp.float32)
    exp = jnp.dot(bt, e_ref[...], preferred_element_type=jnp.float32,
                  precision=lax.Precision.HIGHEST)
    ftile = (jnp.arange(128, dtype=jnp.int32)[None, :] & 7) * 128
    out_ref[...] = exp.astype(jnp.int32) + ftile


def _mlp_body(enc_ref, w1_ref, b1_ref, w2_ref, b2_ref, out_ref):
    e = enc_ref[...]
    h = jnp.dot(e, w1_ref[...], preferred_element_type=jnp.float32) + b1_ref[...]
    h = jnp.where(h > 0, h, 0.01 * h)
    out_ref[...] = jnp.dot(h, w2_ref[...],
                           preferred_element_type=jnp.float32) + b2_ref[...]


_NSLICE = 4
_NS = N // _NSLICE
_TPW = _NS // _NW


@functools.lru_cache(maxsize=None)
def _make_gather_sc():
    @functools.partial(
        pl.kernel,
        mesh=plsc.VectorSubcoreMesh(core_axis_name="c", subcore_axis_name="s"),
        compiler_params=pltpu.CompilerParams(use_tc_tiling_on_sc=False),
        out_type=jax.ShapeDtypeStruct((_NW, _TPW, 128), jnp.float32),
        scratch_types=[
            pltpu.VMEM((_TPW, 128), jnp.int32),
            pltpu.VMEM((_TPW, 128), jnp.float32),
            pltpu.SemaphoreType.DMA,
            pltpu.SemaphoreType.DMA,
        ],
    )
    def _gather_sc(tab_hbm, idx_hbm, out_hbm, idx_v, rows_v, sem0, sem1):
        wid = lax.axis_index("s") * _NC + lax.axis_index("c")
        K = 32

        pltpu.sync_copy(idx_hbm.at[wid], idx_v)

        def fire(j, sem):
            pltpu.make_async_copy(
                tab_hbm.at[idx_v.at[j]], rows_v.at[j], sem
            ).start()

        def drain(j, sem):
            pltpu.make_async_copy(
                tab_hbm.at[idx_v.at[j]], rows_v.at[j], sem
            ).wait()

        for k in range(K):
            fire(k, (sem0, sem1)[k % 2])

        def body(jj, carry):
            j = jj * 2
            drain(j, sem0)
            fire(j + K, sem0)
            drain(j + 1, sem1)
            fire(j + 1 + K, sem1)
            return carry

        lax.fori_loop(0, (_TPW - K) // 2, body, 0)
        for k in range(_TPW - K, _TPW):
            drain(k, (sem0, sem1)[k % 2])
        pltpu.sync_copy(rows_v, out_hbm.at[wid])

    return _gather_sc


def kernel(z, tables, W1, b1, W2, b2):
    HB = 512
    primes = jnp.asarray(_PRIMES).reshape(D, 1)
    expand = jnp.asarray(_EXPAND)

    tab_lin = tables.reshape(L, T // 128, 128, F).transpose(0, 1, 3, 2)
    tab_flat = tab_lin.reshape(L * T * F)

    gather = _make_gather_sc()
    enc_slices = []
    for s in range(_NSLICE):
        idx_s = pl.pallas_call(
            _hash_body,
            grid=(_NS // HB,),
            in_specs=[
                pl.BlockSpec((HB, D), lambda i, s=s: (s * (_NS // HB) + i, 0)),
                pl.BlockSpec((D, 1), lambda i: (0, 0)),
                pl.BlockSpec((L, 128), lambda i: (0, 0)),
            ],
            out_specs=pl.BlockSpec((HB, 128), lambda i: (i, 0)),
            out_shape=jax.ShapeDtypeStruct((_NS, 128), jnp.int32),
        )(z, primes, expand)
        rows_s = gather(tab_flat, idx_s.reshape(_NW, _TPW, 128))
        enc_slices.append(rows_s.reshape(_NS, L * F))

    MB = 1024
    outs = []
    for s in range(_NSLICE):
        out_s = pl.pallas_call(
            _mlp_body,
            grid=(_NS // MB,),
            in_specs=[
                pl.BlockSpec((MB, 128), lambda i: (i, 0)),
                pl.BlockSpec((128, 128), lambda i: (0, 0)),
                pl.BlockSpec((1, 128), lambda i: (0, 0)),
                pl.BlockSpec((128, 256), lambda i: (0, 0)),
                pl.BlockSpec((1, 256), lambda i: (0, 0)),
            ],
            out_specs=pl.BlockSpec((MB, 256), lambda i: (i, 0)),
            out_shape=jax.ShapeDtypeStruct((_NS, 256), jnp.float32),
        )(enc_slices[s], W1, b1.reshape(1, 128), W2, b2.reshape(1, 256))
        outs.append(out_s)
    return jnp.concatenate(outs, axis=0)
